# baseline (speedup 1.0000x reference)
import functools

import jax
import jax.numpy as jnp
from jax import lax
from jax.experimental import pallas as pl
from jax.experimental.pallas import tpu as pltpu
from jax.experimental.pallas import tpu_sc as plsc

_D = 128
_K = 8
_NC = 2
_NS = 16
_NW = _NC * _NS

_HIGH = jax.lax.Precision.HIGHEST


def _qkv_body(f_ref, w_ref, b_ref, o_ref):
    o_ref[...] = (
        jnp.dot(f_ref[...], w_ref[...], precision=_HIGH,
                preferred_element_type=jnp.float32)
        + b_ref[...]
    )


def _qkv_table(feats, w16, b16, bn=800):
    n = feats.shape[0]
    grid = n // bn
    return pl.pallas_call(
        _qkv_body,
        grid=(grid,),
        in_specs=[
            pl.BlockSpec((bn, _D), lambda i: (i, 0)),
            pl.BlockSpec((_D, 16), lambda i: (0, 0)),
            pl.BlockSpec((1, 16), lambda i: (0, 0)),
        ],
        out_specs=pl.BlockSpec((bn, 16), lambda i: (i, 0)),
        out_shape=jax.ShapeDtypeStruct((n, 16), jnp.float32),
    )(feats, w16, b16)


def _vertex_sc(feats, qt, kt, vt, idx_r, e_total, chunk=32):
    per_w = e_total // _NW
    n_chunks = per_w // chunk
    n_pairs = n_chunks // 2
    cs = chunk * _K
    ipc = cs // 128

    mesh = plsc.VectorSubcoreMesh(core_axis_name="c", subcore_axis_name="s")

    buf_types = [
        pltpu.VMEM((cs,), jnp.int32),
        pltpu.VMEM((cs,), jnp.float32),
        pltpu.VMEM((cs + 16,), jnp.float32),
        pltpu.VMEM((cs + 16,), jnp.float32),
        pltpu.VMEM((cs, _D), jnp.float32),
        pltpu.SemaphoreType.DMA,
        pltpu.SemaphoreType.DMA,
    ]

    @functools.partial(
        pl.kernel,
        mesh=mesh,
        out_type=jax.ShapeDtypeStruct((e_total, _D), jnp.float32),
        scratch_types=buf_types + buf_types + [
            pltpu.VMEM((cs + 8,), jnp.float32),
            pltpu.VMEM((chunk, _D), jnp.float32),
        ],
    )
    def sc_kernel(feats_hbm, q_hbm, k_hbm, v_hbm, idx_hbm, out_hbm,
                  idx0, gq0, gk0, gv0, f0, ss0, sf0,
                  idx1, gq1, gk1, gv1, f1, ss1, sf1,
                  d_v, out_v):
        wid = lax.axis_index("s") * _NC + lax.axis_index("c")
        e_base = wid * per_w
        bufs = [(idx0, gq0, gk0, gv0, f0, ss0, sf0),
                (idx1, gq1, gk1, gv1, f1, ss1, sf1)]

        lane = lax.iota(jnp.int32, 16)
        rmod = lane % 8
        sels = [rmod < (8 - m) for m in range(_K)]

        def fire(t, b):
            idx_v, gq, gk, gv, f_v, sem_s, sem_f = bufs[b]
            base = (e_base + t * chunk) * _K
            pltpu.sync_copy(idx_hbm.at[pl.ds(base, cs)], idx_v)
            pltpu.async_copy(q_hbm.at[idx_v], gq, sem_s)
            pltpu.async_copy(k_hbm.at[idx_v], gk.at[pl.ds(8, cs)], sem_s)
            pltpu.async_copy(v_hbm.at[idx_v], gv.at[pl.ds(8, cs)], sem_s)
            for g in range(ipc):
                pltpu.async_copy(
                    feats_hbm.at[idx_v.at[pl.ds(g * 128, 128)]],
                    f_v.at[pl.ds(g * 128, 128)], sem_f)

        def consume(t, b):
            idx_v, gq, gk, gv, f_v, sem_s, sem_f = bufs[b]
            pltpu.make_async_copy(q_hbm.at[pl.ds(0, cs)], gq, sem_s).wait()
            pltpu.make_async_copy(
                q_hbm.at[pl.ds(0, cs)], gk.at[pl.ds(8, cs)], sem_s).wait()
            pltpu.make_async_copy(
                q_hbm.at[pl.ds(0, cs)], gv.at[pl.ds(8, cs)], sem_s).wait()

            for s in range(cs // 16):
                b8 = 8 + s * 16
                q16 = gq[pl.ds(s * 16, 16)]
                s_acc = None
                n_acc = None
                for m in range(1, _K):
                    km = jnp.where(sels[m], gk[pl.ds(b8 + m, 16)],
                                   gk[pl.ds(b8 + m - 8, 16)])
                    vm = jnp.where(sels[m], gv[pl.ds(b8 + m, 16)],
                                   gv[pl.ds(b8 + m - 8, 16)])
                    el = jnp.exp(q16 * km)
                    s_acc = el if s_acc is None else s_acc + el
                    nv = el * vm
                    n_acc = nv if n_acc is None else n_acc + nv
                x = n_acc / s_acc
                em = jnp.exp(-2.0 * x)
                d_v[pl.ds(s * 16, 16)] = (1.0 - em) / (1.0 + em)

            for g in range(ipc):
                pltpu.make_async_copy(
                    feats_hbm.at[pl.ds(0, 128)],
                    f_v.at[pl.ds(g * 128, 128)], sem_f).wait()

            @plsc.parallel_loop(0, chunk, step=4, carry=jnp.int32(0))
            def edge_body(e0, c2):
                for p in range(4):
                    e = e0 + p
                    eb = e * _K
                    dvec = d_v[pl.ds(eb, 16)]
                    ds = [dvec[j] for j in range(_K)]
                    for c in range(_D // 16):
                        csl = pl.ds(c * 16, 16)
                        pr = [ds[j] * f_v[eb + j, csl] for j in range(_K)]
                        out_v[e, csl] = (
                            (pr[0] + pr[1]) + (pr[2] + pr[3])
                        ) + ((pr[4] + pr[5]) + (pr[6] + pr[7]))
                return c2

            pltpu.sync_copy(
                out_v, out_hbm.at[pl.ds(e_base + t * chunk, chunk)])

        fire(0, 0)

        def pair_body(t2, carry):
            t0 = t2 * 2
            fire(t0 + 1, 1)
            consume(t0, 0)

            @pl.when(t2 + 1 < n_pairs)
            def _():
                fire(t0 + 2, 0)

            consume(t0 + 1, 1)
            return carry

        lax.fori_loop(0, n_pairs, pair_body, 0)

    return sc_kernel(feats, qt, kt, vt, idx_r)


def _edge_body(x_ref, m_ref, w1_ref, b1_ref, w2_ref, b2_ref,
               wf_ref, bf_ref, o_ref, *, bb, na1):
    x = x_ref[...]
    h = jnp.maximum(
        jnp.dot(x, w1_ref[...], precision=_HIGH,
                preferred_element_type=jnp.float32) + b1_ref[...], 0.0)
    sc = (jnp.dot(h, w2_ref[...], precision=_HIGH,
                  preferred_element_type=jnp.float32) + b2_ref[...])
    es = jnp.exp(sc) * m_ref[...]
    r = lax.broadcasted_iota(jnp.int32, (bb, bb * na1), 0)
    cg = lax.broadcasted_iota(jnp.int32, (bb, bb * na1), 1) // na1
    seg = (r == cg).astype(jnp.float32)
    numer = jnp.dot(seg, x * es, precision=_HIGH,
                    preferred_element_type=jnp.float32)
    den = jnp.dot(seg, es, precision=_HIGH,
                  preferred_element_type=jnp.float32)
    xb = numer / den
    o_ref[...] = jax.nn.sigmoid(
        jnp.dot(xb, wf_ref[...], precision=_HIGH,
                preferred_element_type=jnp.float32) + bf_ref[...])


def _edge_conv(x2, maskv, fc1_w, fc1_b, fc2_w, fc2_b, fc_w, fc_b,
               b, na1, bb=128):
    grid = b // bb
    hdim = fc1_w.shape[1]
    body = functools.partial(_edge_body, bb=bb, na1=na1)
    return pl.pallas_call(
        body,
        grid=(grid,),
        in_specs=[
            pl.BlockSpec((bb * na1, _D), lambda i: (i, 0)),
            pl.BlockSpec((bb * na1, 1), lambda i: (i, 0)),
            pl.BlockSpec((_D, hdim), lambda i: (0, 0)),
            pl.BlockSpec((1, hdim), lambda i: (0, 0)),
            pl.BlockSpec((hdim, 1), lambda i: (0, 0)),
            pl.BlockSpec((1, 1), lambda i: (0, 0)),
            pl.BlockSpec((_D, 2), lambda i: (0, 0)),
            pl.BlockSpec((1, 2), lambda i: (0, 0)),
        ],
        out_specs=pl.BlockSpec((bb, 2), lambda i: (i, 0)),
        out_shape=jax.ShapeDtypeStruct((b, 2), jnp.float32),
    )(x2, maskv, fc1_w, fc1_b, fc2_w, fc2_b, fc_w, fc_b)


def kernel(ids, feats, edge_dict, adj, epoch,
           wq_w, wq_b, wk_w, wk_b, wv_w, wv_b,
           fc1_w, fc1_b, fc2_w, fc2_b, fc_w, fc_b):
    b = ids.shape[0]
    lo = ids[0]
    ed = lax.dynamic_slice_in_dim(edge_dict, lo, b, axis=0)
    aj = lax.dynamic_slice_in_dim(adj, lo, b, axis=0)
    na1 = 1 + aj.shape[1]
    edges = jnp.concatenate([ed[:, None, :], aj], axis=1)
    e_total = b * na1
    idx_r = edges.reshape(e_total * _K)

    w16 = jnp.concatenate(
        [wq_w, wk_w, wv_w, jnp.zeros((_D, 13), jnp.float32)], axis=1)
    b16 = jnp.concatenate(
        [wq_b, wk_b, wv_b, jnp.zeros((13,), jnp.float32)]).reshape(1, 16)

    qkv = _qkv_table(feats, w16, b16)
    x2 = _vertex_sc(feats, qkv[:, 0], qkv[:, 1], qkv[:, 2],
                    idx_r, e_total)

    m17 = jnp.concatenate([
        jnp.ones((1,), jnp.float32),
        jnp.broadcast_to((epoch >= 5).astype(jnp.float32), (na1 - 1,)),
    ])
    maskv = jnp.tile(m17, b).reshape(e_total, 1)

    return _edge_conv(x2, maskv, fc1_w, fc1_b.reshape(1, -1),
                      fc2_w, fc2_b.reshape(1, 1), fc_w, fc_b.reshape(1, 2),
                      b, na1)


# speedup vs baseline: 1.0020x; 1.0020x over previous
import functools

import jax
import jax.numpy as jnp
from jax import lax
from jax.experimental import pallas as pl
from jax.experimental.pallas import tpu as pltpu
from jax.experimental.pallas import tpu_sc as plsc

_D = 128
_K = 8
_NC = 2
_NS = 16
_NW = _NC * _NS

_HIGH = jax.lax.Precision.HIGHEST


def _qkv_body(f_ref, w_ref, b_ref, o_ref):
    o_ref[...] = (
        jnp.dot(f_ref[...], w_ref[...], precision=_HIGH,
                preferred_element_type=jnp.float32)
        + b_ref[...]
    )


def _qkv_table(feats, w16, b16, bn=800):
    n = feats.shape[0]
    grid = n // bn
    return pl.pallas_call(
        _qkv_body,
        grid=(grid,),
        in_specs=[
            pl.BlockSpec((bn, _D), lambda i: (i, 0)),
            pl.BlockSpec((_D, 16), lambda i: (0, 0)),
            pl.BlockSpec((1, 16), lambda i: (0, 0)),
        ],
        out_specs=pl.BlockSpec((bn, 16), lambda i: (i, 0)),
        out_shape=jax.ShapeDtypeStruct((n, 16), jnp.float32),
    )(feats, w16, b16)


def _vertex_sc(feats, qt, kt, vt, idx_r, e_total, chunk=32):
    per_w = e_total // _NW
    n_chunks = per_w // chunk
    n_pairs = n_chunks // 2
    cs = chunk * _K
    ipc = cs // 128

    mesh = plsc.VectorSubcoreMesh(core_axis_name="c", subcore_axis_name="s")

    buf_types = [
        pltpu.VMEM((cs,), jnp.int32),
        pltpu.VMEM((cs,), jnp.float32),
        pltpu.VMEM((cs + 16,), jnp.float32),
        pltpu.VMEM((cs + 16,), jnp.float32),
        pltpu.VMEM((cs, _D), jnp.float32),
        pltpu.SemaphoreType.DMA,
        pltpu.SemaphoreType.DMA,
    ]

    @functools.partial(
        pl.kernel,
        mesh=mesh,
        out_type=jax.ShapeDtypeStruct((e_total, _D), jnp.float32),
        scratch_types=buf_types + buf_types + [
            pltpu.VMEM((cs + 8,), jnp.float32),
            pltpu.VMEM((chunk, _D), jnp.float32),
        ],
    )
    def sc_kernel(feats_hbm, q_hbm, k_hbm, v_hbm, idx_hbm, out_hbm,
                  idx0, gq0, gk0, gv0, f0, ss0, sf0,
                  idx1, gq1, gk1, gv1, f1, ss1, sf1,
                  d_v, out_v):
        wid = lax.axis_index("s") * _NC + lax.axis_index("c")
        e_base = wid * per_w
        bufs = [(idx0, gq0, gk0, gv0, f0, ss0, sf0),
                (idx1, gq1, gk1, gv1, f1, ss1, sf1)]

        lane = lax.iota(jnp.int32, 16)
        rmod = lane % 8
        sels = [rmod < (8 - m) for m in range(_K)]

        def fire(t, b):
            idx_v, gq, gk, gv, f_v, sem_s, sem_f = bufs[b]
            base = (e_base + t * chunk) * _K
            pltpu.sync_copy(idx_hbm.at[pl.ds(base, cs)], idx_v)
            pltpu.async_copy(q_hbm.at[idx_v], gq, sem_s)
            pltpu.async_copy(k_hbm.at[idx_v], gk.at[pl.ds(8, cs)], sem_s)
            pltpu.async_copy(v_hbm.at[idx_v], gv.at[pl.ds(8, cs)], sem_s)
            for g in range(ipc):
                pltpu.async_copy(
                    feats_hbm.at[idx_v.at[pl.ds(g * 128, 128)]],
                    f_v.at[pl.ds(g * 128, 128)], sem_f)

        def consume(t, b):
            idx_v, gq, gk, gv, f_v, sem_s, sem_f = bufs[b]
            pltpu.make_async_copy(q_hbm.at[pl.ds(0, cs)], gq, sem_s).wait()
            pltpu.make_async_copy(
                q_hbm.at[pl.ds(0, cs)], gk.at[pl.ds(8, cs)], sem_s).wait()
            pltpu.make_async_copy(
                q_hbm.at[pl.ds(0, cs)], gv.at[pl.ds(8, cs)], sem_s).wait()

            for s in range(cs // 16):
                b8 = 8 + s * 16
                q16 = gq[pl.ds(s * 16, 16)]
                s_acc = None
                n_acc = None
                for m in range(1, _K):
                    km = jnp.where(sels[m], gk[pl.ds(b8 + m, 16)],
                                   gk[pl.ds(b8 + m - 8, 16)])
                    vm = jnp.where(sels[m], gv[pl.ds(b8 + m, 16)],
                                   gv[pl.ds(b8 + m - 8, 16)])
                    el = jnp.exp(q16 * km)
                    s_acc = el if s_acc is None else s_acc + el
                    nv = el * vm
                    n_acc = nv if n_acc is None else n_acc + nv
                x = n_acc / s_acc
                em = jnp.exp(-2.0 * x)
                d_v[pl.ds(s * 16, 16)] = (1.0 - em) / (1.0 + em)

            for g in range(ipc):
                pltpu.make_async_copy(
                    feats_hbm.at[pl.ds(0, 128)],
                    f_v.at[pl.ds(g * 128, 128)], sem_f).wait()

            @plsc.parallel_loop(0, chunk, step=4, carry=jnp.int32(0))
            def edge_body(e0, c2):
                dss = []
                for p in range(4):
                    dvec = d_v[pl.ds((e0 + p) * _K, 16)]
                    dss.append([dvec[j] for j in range(_K)])
                for c in range(_D // 16):
                    csl = pl.ds(c * 16, 16)
                    for p in range(4):
                        e = e0 + p
                        eb = e * _K
                        ds = dss[p]
                        pr = [ds[j] * f_v[eb + j, csl] for j in range(_K)]
                        out_v[e, csl] = (
                            (pr[0] + pr[1]) + (pr[2] + pr[3])
                        ) + ((pr[4] + pr[5]) + (pr[6] + pr[7]))
                return c2

            pltpu.sync_copy(
                out_v, out_hbm.at[pl.ds(e_base + t * chunk, chunk)])

        fire(0, 0)

        def pair_body(t2, carry):
            t0 = t2 * 2
            fire(t0 + 1, 1)
            consume(t0, 0)

            @pl.when(t2 + 1 < n_pairs)
            def _():
                fire(t0 + 2, 0)

            consume(t0 + 1, 1)
            return carry

        lax.fori_loop(0, n_pairs, pair_body, 0)

    return sc_kernel(feats, qt, kt, vt, idx_r)


def _edge_body(x_ref, m_ref, w1_ref, b1_ref, w2_ref, b2_ref,
               wf_ref, bf_ref, o_ref, *, bb, na1):
    x = x_ref[...]
    h = jnp.maximum(
        jnp.dot(x, w1_ref[...], precision=_HIGH,
                preferred_element_type=jnp.float32) + b1_ref[...], 0.0)
    sc = (jnp.dot(h, w2_ref[...], precision=_HIGH,
                  preferred_element_type=jnp.float32) + b2_ref[...])
    es = jnp.exp(sc) * m_ref[...]
    r = lax.broadcasted_iota(jnp.int32, (bb, bb * na1), 0)
    cg = lax.broadcasted_iota(jnp.int32, (bb, bb * na1), 1) // na1
    seg = (r == cg).astype(jnp.float32)
    numer = jnp.dot(seg, x * es, precision=_HIGH,
                    preferred_element_type=jnp.float32)
    den = jnp.dot(seg, es, precision=_HIGH,
                  preferred_element_type=jnp.float32)
    xb = numer / den
    o_ref[...] = jax.nn.sigmoid(
        jnp.dot(xb, wf_ref[...], precision=_HIGH,
                preferred_element_type=jnp.float32) + bf_ref[...])


def _edge_conv(x2, maskv, fc1_w, fc1_b, fc2_w, fc2_b, fc_w, fc_b,
               b, na1, bb=128):
    grid = b // bb
    hdim = fc1_w.shape[1]
    body = functools.partial(_edge_body, bb=bb, na1=na1)
    return pl.pallas_call(
        body,
        grid=(grid,),
        in_specs=[
            pl.BlockSpec((bb * na1, _D), lambda i: (i, 0)),
            pl.BlockSpec((bb * na1, 1), lambda i: (i, 0)),
            pl.BlockSpec((_D, hdim), lambda i: (0, 0)),
            pl.BlockSpec((1, hdim), lambda i: (0, 0)),
            pl.BlockSpec((hdim, 1), lambda i: (0, 0)),
            pl.BlockSpec((1, 1), lambda i: (0, 0)),
            pl.BlockSpec((_D, 2), lambda i: (0, 0)),
            pl.BlockSpec((1, 2), lambda i: (0, 0)),
        ],
        out_specs=pl.BlockSpec((bb, 2), lambda i: (i, 0)),
        out_shape=jax.ShapeDtypeStruct((b, 2), jnp.float32),
    )(x2, maskv, fc1_w, fc1_b, fc2_w, fc2_b, fc_w, fc_b)


def kernel(ids, feats, edge_dict, adj, epoch,
           wq_w, wq_b, wk_w, wk_b, wv_w, wv_b,
           fc1_w, fc1_b, fc2_w, fc2_b, fc_w, fc_b):
    b = ids.shape[0]
    lo = ids[0]
    ed = lax.dynamic_slice_in_dim(edge_dict, lo, b, axis=0)
    aj = lax.dynamic_slice_in_dim(adj, lo, b, axis=0)
    na1 = 1 + aj.shape[1]
    edges = jnp.concatenate([ed[:, None, :], aj], axis=1)
    e_total = b * na1
    idx_r = edges.reshape(e_total * _K)

    w16 = jnp.concatenate(
        [wq_w, wk_w, wv_w, jnp.zeros((_D, 13), jnp.float32)], axis=1)
    b16 = jnp.concatenate(
        [wq_b, wk_b, wv_b, jnp.zeros((13,), jnp.float32)]).reshape(1, 16)

    qkv = _qkv_table(feats, w16, b16)
    x2 = _vertex_sc(feats, qkv[:, 0], qkv[:, 1], qkv[:, 2],
                    idx_r, e_total)

    m17 = jnp.concatenate([
        jnp.ones((1,), jnp.float32),
        jnp.broadcast_to((epoch >= 5).astype(jnp.float32), (na1 - 1,)),
    ])
    maskv = jnp.tile(m17, b).reshape(e_total, 1)

    return _edge_conv(x2, maskv, fc1_w, fc1_b.reshape(1, -1),
                      fc2_w, fc2_b.reshape(1, 1), fc_w, fc_b.reshape(1, 2),
                      b, na1)
